# Initial kernel scaffold; baseline (speedup 1.0000x reference)
import jax
import jax.numpy as jnp
from jax.experimental import pallas as pl


def kernel(x, edge_index, W, b):
    raise NotImplementedError("write your pallas kernel here")


# speedup vs baseline: 2.7885x; 2.7885x over previous
import functools
import math

import jax
import jax.numpy as jnp
from jax import lax
from jax.experimental import pallas as pl
from jax.experimental.pallas import tpu as pltpu
from jax.experimental.pallas import tpu_sc as plsc

_NC = 2
_NS = 16
_NW = _NC * _NS
_C = 128
_BLK = 80


def _sc_aggregate(h, src3, dst3, n_pad, ch):
    d = h.shape[1]
    rows_per_sub = n_pad // _NS
    full, rem = divmod(rows_per_sub, _C)

    mesh = plsc.VectorSubcoreMesh(core_axis_name="c", subcore_axis_name="s")

    @functools.partial(
        pl.kernel,
        out_type=jax.ShapeDtypeStruct((_NC * n_pad, d), jnp.float32),
        mesh=mesh,
        scratch_types=[
            pltpu.VMEM((ch, _C), jnp.int32),
            pltpu.VMEM((ch, _C), jnp.int32),
            pltpu.VMEM((_C, d), jnp.float32),
            pltpu.VMEM_SHARED((n_pad, d), jnp.float32),
            pltpu.SemaphoreType.DMA,
        ],
    )
    def agg(h_hbm, src_hbm, dst_hbm, out_hbm, src_v, dst_v, rows_v, acc_sh, sem):
        c = lax.axis_index("c")
        s = lax.axis_index("s")
        w = c * _NS + s

        zval = jnp.zeros((16,), jnp.float32)

        def zero_body(i, carry):
            for k in range(d // 16):
                rows_v[i, pl.ds(k * 16, 16)] = zval
            return carry

        lax.fori_loop(0, _C, zero_body, 0)

        base = s * rows_per_sub
        for t in range(full):
            pltpu.sync_copy(rows_v, acc_sh.at[pl.ds(base + t * _C, _C)])
        if rem:
            pltpu.sync_copy(rows_v.at[pl.ds(0, rem)],
                            acc_sh.at[pl.ds(base + full * _C, rem)])
        plsc.subcore_barrier()

        pltpu.sync_copy(src_hbm.at[w], src_v)
        pltpu.sync_copy(dst_hbm.at[w], dst_v)

        def chunk_body(j, carry):
            pltpu.async_copy(h_hbm.at[src_v.at[j]], rows_v, sem).wait()
            pltpu.sync_copy(rows_v, acc_sh.at[dst_v.at[j]], add=True)
            return carry

        lax.fori_loop(0, ch, chunk_body, 0)
        plsc.subcore_barrier()

        pltpu.sync_copy(acc_sh.at[pl.ds(base, rows_per_sub)],
                        out_hbm.at[pl.ds(c * n_pad + base, rows_per_sub)])

    return agg(h, src3, dst3)


def _tc_transform(parts, w_mat, b_row, n, n_pad):
    d = w_mat.shape[0]
    nb = n // _BLK
    off = n_pad // _BLK

    def body(p0_ref, p1_ref, w_ref, b_ref, o_ref):
        a = p0_ref[...] + p1_ref[...]
        y = jnp.dot(a, w_ref[...], preferred_element_type=jnp.float32) + b_ref[...]
        o_ref[...] = y * (1.0 / (1.0 + jnp.exp(-y)))

    return pl.pallas_call(
        body,
        grid=(nb,),
        in_specs=[
            pl.BlockSpec((_BLK, d), lambda i: (i, 0)),
            pl.BlockSpec((_BLK, d), lambda i: (i + off, 0)),
            pl.BlockSpec((d, d), lambda i: (0, 0)),
            pl.BlockSpec((1, d), lambda i: (0, 0)),
        ],
        out_specs=pl.BlockSpec((_BLK, d), lambda i: (i, 0)),
        out_shape=jax.ShapeDtypeStruct((n, d), jnp.float32),
    )(parts, parts, w_mat, b_row)


def kernel(x, edge_index, W, b):
    n, d = x.shape
    e = edge_index.shape[1]
    assert n % _BLK == 0
    lcm = _BLK * 128 // math.gcd(_BLK, 128)
    n_pad = ((n // lcm) + 1) * lcm

    ch = -(-e // (_NW * _C * 8)) * 8
    e_pad = _NW * ch * _C
    pad = e_pad - e
    src = edge_index[0]
    dst = edge_index[1]
    if pad:
        src = jnp.concatenate([src, jnp.zeros((pad,), jnp.int32)])
        dst = jnp.concatenate([dst, jnp.full((pad,), n, jnp.int32)])
    src3 = src.reshape(_NW, ch, _C)
    dst3 = dst.reshape(_NW, ch, _C)
    b_row = b.reshape(1, d)

    h = x
    for _ in range(2):
        parts = _sc_aggregate(h, src3, dst3, n_pad, ch)
        h = _tc_transform(parts, W, b_row, n, n_pad)
    return h


# speedup vs baseline: 2.9491x; 1.0576x over previous
import functools
import math

import jax
import jax.numpy as jnp
from jax import lax
from jax.experimental import pallas as pl
from jax.experimental.pallas import tpu as pltpu
from jax.experimental.pallas import tpu_sc as plsc

_NC = 2
_NS = 16
_NW = _NC * _NS
_C = 128
_G = 16
_BLK = 80


def _sc_aggregate(h, src3, dst3, n_pad, ch):
    d = h.shape[1]
    rows_per_sub = n_pad // _NS
    full, rem = divmod(rows_per_sub, _C)

    mesh = plsc.VectorSubcoreMesh(core_axis_name="c", subcore_axis_name="s")

    @functools.partial(
        pl.kernel,
        out_type=jax.ShapeDtypeStruct((_NC * n_pad, d), jnp.float32),
        mesh=mesh,
        scratch_types=[
            pltpu.VMEM((_G, _C), jnp.int32),
            pltpu.VMEM((_G, _C), jnp.int32),
            pltpu.VMEM((_C, d), jnp.float32),
            pltpu.VMEM((_C, d), jnp.float32),
            pltpu.VMEM_SHARED((n_pad, d), jnp.float32),
            pltpu.SemaphoreType.DMA,
            pltpu.SemaphoreType.DMA,
        ],
    )
    def agg(h_hbm, src_hbm, dst_hbm, out_hbm, sidx, didx,
            rows0, rows1, acc_sh, sem0, sem1):
        c = lax.axis_index("c")
        s = lax.axis_index("s")
        w = c * _NS + s

        zval = jnp.zeros((16,), jnp.float32)

        def zero_body(i, carry):
            for k in range(d // 16):
                rows0[i, pl.ds(k * 16, 16)] = zval
            return carry

        lax.fori_loop(0, _C, zero_body, 0)

        base = s * rows_per_sub
        for t in range(full):
            pltpu.sync_copy(rows0, acc_sh.at[pl.ds(base + t * _C, _C)])
        if rem:
            pltpu.sync_copy(rows0.at[pl.ds(0, rem)],
                            acc_sh.at[pl.ds(base + full * _C, rem)])

        plsc.subcore_barrier()

        ng = ch // _G

        def group_body(g, carry):
            cbase = g * _G
            pltpu.sync_copy(src_hbm.at[w, pl.ds(cbase, _G)], sidx)
            pltpu.sync_copy(dst_hbm.at[w, pl.ds(cbase, _G)], didx)
            bufs = ((rows0, sem0), (rows1, sem1))
            cp = pltpu.async_copy(h_hbm.at[sidx.at[0]], rows0, sem0)
            for k in range(_G):
                cp.wait()
                if k + 1 < _G:
                    buf, sem = bufs[(k + 1) % 2]
                    cp = pltpu.async_copy(h_hbm.at[sidx.at[k + 1]], buf, sem)
                pltpu.sync_copy(bufs[k % 2][0], acc_sh.at[didx.at[k]], add=True)
            return carry

        lax.fori_loop(0, ng, group_body, 0)
        plsc.subcore_barrier()

        pltpu.sync_copy(acc_sh.at[pl.ds(base, rows_per_sub)],
                        out_hbm.at[pl.ds(c * n_pad + base, rows_per_sub)])

    return agg(h, src3, dst3)


def _tc_transform(parts, w_mat, b_row, n, n_pad):
    d = w_mat.shape[0]
    nb = n // _BLK
    off = n_pad // _BLK

    def body(p0_ref, p1_ref, w_ref, b_ref, o_ref):
        a = p0_ref[...] + p1_ref[...]
        y = jnp.dot(a, w_ref[...], preferred_element_type=jnp.float32) + b_ref[...]
        o_ref[...] = y * (1.0 / (1.0 + jnp.exp(-y)))

    return pl.pallas_call(
        body,
        grid=(nb,),
        in_specs=[
            pl.BlockSpec((_BLK, d), lambda i: (i, 0)),
            pl.BlockSpec((_BLK, d), lambda i: (i + off, 0)),
            pl.BlockSpec((d, d), lambda i: (0, 0)),
            pl.BlockSpec((1, d), lambda i: (0, 0)),
        ],
        out_specs=pl.BlockSpec((_BLK, d), lambda i: (i, 0)),
        out_shape=jax.ShapeDtypeStruct((n, d), jnp.float32),
    )(parts, parts, w_mat, b_row)


def kernel(x, edge_index, W, b):
    n, d = x.shape
    e = edge_index.shape[1]
    assert n % _BLK == 0
    lcm = _BLK * 128 // math.gcd(_BLK, 128)
    n_pad = ((n // lcm) + 1) * lcm

    ch = -(-e // (_NW * _C * _G)) * _G
    e_pad = _NW * ch * _C
    pad = e_pad - e
    src = edge_index[0]
    dst = edge_index[1]
    if pad:
        src = jnp.concatenate([src, jnp.zeros((pad,), jnp.int32)])
        dst = jnp.concatenate([dst, jnp.full((pad,), n, jnp.int32)])
    src3 = src.reshape(_NW, ch, _C)
    dst3 = dst.reshape(_NW, ch, _C)
    b_row = b.reshape(1, d)

    h = x
    for _ in range(2):
        parts = _sc_aggregate(h, src3, dst3, n_pad, ch)
        h = _tc_transform(parts, W, b_row, n, n_pad)
    return h
